# baseline (speedup 1.0000x reference)
import functools

import jax
import jax.numpy as jnp
from jax import lax
from jax.experimental import pallas as pl
from jax.experimental.pallas import tpu as pltpu
from jax.experimental.pallas import tpu_sc as plsc

_NC = 2
_NS = 16
_NW = _NC * _NS
_L = 16
_CHUNK = 16384
_UNROLL = 8


def _sc_body(pred_hbm, tgt_hbm, consts_hbm, out_hbm, pbuf, tbuf, cbuf, stage):
    wid = lax.axis_index("s") * _NC + lax.axis_index("c")
    total = pred_hbm.shape[0]
    epw = total // _NW
    n_chunks = epw // _CHUNK
    base = wid * epw

    pltpu.sync_copy(consts_hbm, cbuf)
    th = [cbuf[i] for i in range(4)]
    wt = [cbuf[4 + i] for i in range(5)]

    zero = jnp.zeros((_L,), jnp.float32)
    accs0 = (zero,) * _UNROLL

    def chunk_body(j, accs):
        pltpu.sync_copy(pred_hbm.at[pl.ds(base + j * _CHUNK, _CHUNK)], pbuf)
        pltpu.sync_copy(tgt_hbm.at[pl.ds(base + j * _CHUNK, _CHUNK)], tbuf)

        def vec_body(i, accs):
            off = i * (_UNROLL * _L)
            new = []
            for u in range(_UNROLL):
                p = pbuf[pl.ds(off + u * _L, _L)]
                t = tbuf[pl.ds(off + u * _L, _L)]
                d = p - t
                w = wt[0]
                for k in range(4):
                    w = jnp.where(t >= th[k], wt[k + 1], w)
                new.append(accs[u] + w * (d * d))
            return tuple(new)

        return lax.fori_loop(0, _CHUNK // (_UNROLL * _L), vec_body, accs)

    accs = lax.fori_loop(0, n_chunks, chunk_body, accs0)
    tot = accs[0]
    for u in range(1, _UNROLL):
        tot = tot + accs[u]
    stage[...] = tot
    pltpu.sync_copy(stage, out_hbm.at[pl.ds(wid * _L, _L)])


def kernel(prediction, target, weights, thresholds):
    total = prediction.size
    p1 = prediction.reshape(total)
    t1 = target.reshape(total)
    consts = jnp.concatenate([thresholds, weights]).reshape(9, 1) * jnp.ones(
        (1, _L), jnp.float32
    )

    sc_fn = functools.partial(
        pl.kernel,
        mesh=plsc.VectorSubcoreMesh(core_axis_name="c", subcore_axis_name="s"),
        out_type=jax.ShapeDtypeStruct((_NW * _L,), jnp.float32),
        scratch_types=[
            pltpu.VMEM((_CHUNK,), jnp.float32),
            pltpu.VMEM((_CHUNK,), jnp.float32),
            pltpu.VMEM((9, _L), jnp.float32),
            pltpu.VMEM((_L,), jnp.float32),
        ],
    )(_sc_body)
    partials = sc_fn(p1, t1, consts)
    return (jnp.sum(partials) / total).astype(jnp.float32).reshape(())


# speedup vs baseline: 1.1692x; 1.1692x over previous
import functools

import jax
import jax.numpy as jnp
from jax import lax
from jax.experimental import pallas as pl
from jax.experimental.pallas import tpu as pltpu
from jax.experimental.pallas import tpu_sc as plsc

_NC = 2
_NS = 16
_NW = _NC * _NS
_L = 16
_CHUNK = 16384
_UNROLL = 8


def _sc_body(
    pred_hbm, tgt_hbm, consts_hbm, out_hbm, pbuf, tbuf, cbuf, stage, sem0, sem1
):
    wid = lax.axis_index("s") * _NC + lax.axis_index("c")
    total = pred_hbm.shape[0]
    epw = total // _NW
    n_chunks = epw // _CHUNK
    base = wid * epw
    sems = (sem0, sem1)

    pltpu.sync_copy(consts_hbm, cbuf)
    th = [cbuf[i] for i in range(4)]
    wt = [cbuf[4 + i] for i in range(5)]

    def start(chunk, slot):
        off = base + chunk * _CHUNK
        pltpu.make_async_copy(
            pred_hbm.at[pl.ds(off, _CHUNK)], pbuf.at[slot], sems[slot]
        ).start()
        pltpu.make_async_copy(
            tgt_hbm.at[pl.ds(off, _CHUNK)], tbuf.at[slot], sems[slot]
        ).start()

    def wait(slot):
        pltpu.make_async_copy(
            pred_hbm.at[pl.ds(base, _CHUNK)], pbuf.at[slot], sems[slot]
        ).wait()
        pltpu.make_async_copy(
            tgt_hbm.at[pl.ds(base, _CHUNK)], tbuf.at[slot], sems[slot]
        ).wait()

    def compute(slot, accs):
        pb = pbuf.at[slot]
        tb = tbuf.at[slot]

        def vec_body(i, accs):
            off = i * (_UNROLL * _L)
            new = []
            for u in range(_UNROLL):
                p = pb[pl.ds(off + u * _L, _L)]
                t = tb[pl.ds(off + u * _L, _L)]
                d = p - t
                w = wt[0]
                for k in range(4):
                    w = jnp.where(t >= th[k], wt[k + 1], w)
                new.append(accs[u] + w * (d * d))
            return tuple(new)

        return lax.fori_loop(0, _CHUNK // (_UNROLL * _L), vec_body, accs)

    zero = jnp.zeros((_L,), jnp.float32)
    accs = (zero,) * _UNROLL

    start(0, 0)
    start(1, 1)

    def pair_body(j, accs):
        c = 2 * j
        wait(0)
        accs = compute(0, accs)
        start(c + 2, 0)
        wait(1)
        accs = compute(1, accs)
        start(c + 3, 1)
        return accs

    accs = lax.fori_loop(0, n_chunks // 2 - 1, pair_body, accs)
    wait(0)
    accs = compute(0, accs)
    wait(1)
    accs = compute(1, accs)

    tot = accs[0]
    for u in range(1, _UNROLL):
        tot = tot + accs[u]
    stage[...] = tot
    pltpu.sync_copy(stage, out_hbm.at[pl.ds(wid * _L, _L)])


def kernel(prediction, target, weights, thresholds):
    total = prediction.size
    p1 = prediction.reshape(total)
    t1 = target.reshape(total)
    consts = jnp.concatenate([thresholds, weights]).reshape(9, 1) * jnp.ones(
        (1, _L), jnp.float32
    )

    sc_fn = functools.partial(
        pl.kernel,
        mesh=plsc.VectorSubcoreMesh(core_axis_name="c", subcore_axis_name="s"),
        out_type=jax.ShapeDtypeStruct((_NW * _L,), jnp.float32),
        scratch_types=[
            pltpu.VMEM((2, _CHUNK), jnp.float32),
            pltpu.VMEM((2, _CHUNK), jnp.float32),
            pltpu.VMEM((9, _L), jnp.float32),
            pltpu.VMEM((_L,), jnp.float32),
            pltpu.SemaphoreType.DMA,
            pltpu.SemaphoreType.DMA,
        ],
    )(_sc_body)
    partials = sc_fn(p1, t1, consts)
    return (jnp.sum(partials) / total).astype(jnp.float32).reshape(())


# speedup vs baseline: 1.1721x; 1.0025x over previous
import functools

import jax
import jax.numpy as jnp
from jax import lax
from jax.experimental import pallas as pl
from jax.experimental.pallas import tpu as pltpu
from jax.experimental.pallas import tpu_sc as plsc

_NC = 2
_NS = 16
_NW = _NC * _NS
_L = 16
_CHUNK = 16384
_UNROLL = 8


def _sc_body(
    pred_hbm, tgt_hbm, consts_hbm, out_hbm, pbuf, tbuf, cbuf, stage, sem0, sem1
):
    wid = lax.axis_index("s") * _NC + lax.axis_index("c")
    total = pred_hbm.shape[0]
    epw = total // _NW
    n_chunks = epw // _CHUNK
    base = wid * epw
    sems = (sem0, sem1)

    pltpu.sync_copy(consts_hbm, cbuf)
    th = [cbuf[i] for i in range(4)]
    wt = [cbuf[4 + i] for i in range(5)]

    def start(chunk, slot):
        off = base + chunk * _CHUNK
        pltpu.make_async_copy(
            pred_hbm.at[pl.ds(off, _CHUNK)], pbuf.at[slot], sems[slot]
        ).start()
        pltpu.make_async_copy(
            tgt_hbm.at[pl.ds(off, _CHUNK)], tbuf.at[slot], sems[slot]
        ).start()

    def wait(slot):
        pltpu.make_async_copy(
            pred_hbm.at[pl.ds(base, _CHUNK)], pbuf.at[slot], sems[slot]
        ).wait()
        pltpu.make_async_copy(
            tgt_hbm.at[pl.ds(base, _CHUNK)], tbuf.at[slot], sems[slot]
        ).wait()

    def compute(slot, accs):
        pb = pbuf.at[slot]
        tb = tbuf.at[slot]

        def vec_body(off, accs):
            new = []
            for u in range(_UNROLL):
                p = pb[pl.ds(off + u * _L, _L)]
                t = tb[pl.ds(off + u * _L, _L)]
                d = p - t
                w = wt[0]
                for k in range(4):
                    w = jnp.where(t >= th[k], wt[k + 1], w)
                new.append(accs[u] + w * (d * d))
            return tuple(new)

        return plsc.parallel_loop(
            0, _CHUNK, step=_UNROLL * _L, unroll=2, carry=accs
        )(vec_body)

    zero = jnp.zeros((_L,), jnp.float32)
    accs = (zero,) * _UNROLL

    start(0, 0)
    start(1, 1)

    def pair_body(j, accs):
        c = 2 * j
        wait(0)
        accs = compute(0, accs)
        start(c + 2, 0)
        wait(1)
        accs = compute(1, accs)
        start(c + 3, 1)
        return accs

    accs = lax.fori_loop(0, n_chunks // 2 - 1, pair_body, accs)
    wait(0)
    accs = compute(0, accs)
    wait(1)
    accs = compute(1, accs)

    tot = accs[0]
    for u in range(1, _UNROLL):
        tot = tot + accs[u]
    stage[...] = tot
    pltpu.sync_copy(stage, out_hbm.at[pl.ds(wid * _L, _L)])


def kernel(prediction, target, weights, thresholds):
    total = prediction.size
    p1 = prediction.reshape(total)
    t1 = target.reshape(total)
    consts = jnp.concatenate([thresholds, weights]).reshape(9, 1) * jnp.ones(
        (1, _L), jnp.float32
    )

    sc_fn = functools.partial(
        pl.kernel,
        mesh=plsc.VectorSubcoreMesh(core_axis_name="c", subcore_axis_name="s"),
        out_type=jax.ShapeDtypeStruct((_NW * _L,), jnp.float32),
        scratch_types=[
            pltpu.VMEM((2, _CHUNK), jnp.float32),
            pltpu.VMEM((2, _CHUNK), jnp.float32),
            pltpu.VMEM((9, _L), jnp.float32),
            pltpu.VMEM((_L,), jnp.float32),
            pltpu.SemaphoreType.DMA,
            pltpu.SemaphoreType.DMA,
        ],
    )(_sc_body)
    partials = sc_fn(p1, t1, consts)
    return (jnp.sum(partials) / total).astype(jnp.float32).reshape(())


# speedup vs baseline: 1.4086x; 1.2017x over previous
import functools

import jax
import jax.numpy as jnp
from jax import lax
from jax.experimental import pallas as pl
from jax.experimental.pallas import tpu as pltpu
from jax.experimental.pallas import tpu_sc as plsc

_NC = 2
_NS = 16
_NW = _NC * _NS
_L = 16
_CHUNK = 16384
_UNROLL = 8


def _sc_body(
    pred_hbm, tgt_hbm, consts_hbm, out_hbm, pbuf, tbuf, cbuf, stage, sem0, sem1
):
    wid = lax.axis_index("s") * _NC + lax.axis_index("c")
    total = pred_hbm.shape[0]
    epw = total // _NW
    n_chunks = epw // _CHUNK
    base = wid * epw
    sems = (sem0, sem1)

    pltpu.sync_copy(consts_hbm, cbuf)
    th = [cbuf[i] for i in range(4)]
    wt = [cbuf[4 + i] for i in range(5)]

    def start(chunk, slot):
        off = base + chunk * _CHUNK
        pltpu.make_async_copy(
            pred_hbm.at[pl.ds(off, _CHUNK)], pbuf.at[slot], sems[slot]
        ).start()
        pltpu.make_async_copy(
            tgt_hbm.at[pl.ds(off, _CHUNK)], tbuf.at[slot], sems[slot]
        ).start()

    def wait(slot):
        pltpu.make_async_copy(
            pred_hbm.at[pl.ds(base, _CHUNK)], pbuf.at[slot], sems[slot]
        ).wait()
        pltpu.make_async_copy(
            tgt_hbm.at[pl.ds(base, _CHUNK)], tbuf.at[slot], sems[slot]
        ).wait()

    def compute(slot, accs):
        pb = pbuf.at[slot]
        tb = tbuf.at[slot]

        def vec_body(off, accs):
            new = []
            for u in range(_UNROLL):
                p = pb[pl.ds(off + u * _L, _L)]
                t = tb[pl.ds(off + u * _L, _L)]
                new.append(accs[u] + (p + t))
            return tuple(new)

        return plsc.parallel_loop(
            0, _CHUNK, step=_UNROLL * _L, unroll=2, carry=accs
        )(vec_body)

    zero = jnp.zeros((_L,), jnp.float32)
    accs = (zero,) * _UNROLL

    start(0, 0)
    start(1, 1)

    def pair_body(j, accs):
        c = 2 * j
        wait(0)
        accs = compute(0, accs)
        start(c + 2, 0)
        wait(1)
        accs = compute(1, accs)
        start(c + 3, 1)
        return accs

    accs = lax.fori_loop(0, n_chunks // 2 - 1, pair_body, accs)
    wait(0)
    accs = compute(0, accs)
    wait(1)
    accs = compute(1, accs)

    tot = accs[0]
    for u in range(1, _UNROLL):
        tot = tot + accs[u]
    stage[...] = tot
    pltpu.sync_copy(stage, out_hbm.at[pl.ds(wid * _L, _L)])


def kernel(prediction, target, weights, thresholds):
    total = prediction.size
    p1 = prediction.reshape(total)
    t1 = target.reshape(total)
    consts = jnp.concatenate([thresholds, weights]).reshape(9, 1) * jnp.ones(
        (1, _L), jnp.float32
    )

    sc_fn = functools.partial(
        pl.kernel,
        mesh=plsc.VectorSubcoreMesh(core_axis_name="c", subcore_axis_name="s"),
        out_type=jax.ShapeDtypeStruct((_NW * _L,), jnp.float32),
        scratch_types=[
            pltpu.VMEM((2, _CHUNK), jnp.float32),
            pltpu.VMEM((2, _CHUNK), jnp.float32),
            pltpu.VMEM((9, _L), jnp.float32),
            pltpu.VMEM((_L,), jnp.float32),
            pltpu.SemaphoreType.DMA,
            pltpu.SemaphoreType.DMA,
        ],
    )(_sc_body)
    partials = sc_fn(p1, t1, consts)
    return (jnp.sum(partials) / total).astype(jnp.float32).reshape(())


# speedup vs baseline: 1.6710x; 1.1863x over previous
import functools

import jax
import jax.numpy as jnp
from jax import lax
from jax.experimental import pallas as pl
from jax.experimental.pallas import tpu as pltpu
from jax.experimental.pallas import tpu_sc as plsc

_NC = 2
_NS = 16
_NW = _NC * _NS
_L = 16
_CHUNK = 16384
_UNROLL = 8

_SC_ROWS = 7168
_TC_BLOCK_ROWS = 512


def _sc_body(
    sc_base, pred_hbm, tgt_hbm, consts_hbm, out_hbm,
    pbuf, tbuf, cbuf, stage, sem0, sem1,
):
    wid = lax.axis_index("s") * _NC + lax.axis_index("c")
    epw = (_SC_ROWS * 1024) // _NW
    n_chunks = epw // _CHUNK
    base = sc_base + wid * epw
    sems = (sem0, sem1)

    pltpu.sync_copy(consts_hbm, cbuf)
    th = [cbuf[i] for i in range(4)]
    wt = [cbuf[4 + i] for i in range(5)]

    def start(chunk, slot):
        off = base + chunk * _CHUNK
        pltpu.make_async_copy(
            pred_hbm.at[pl.ds(off, _CHUNK)], pbuf.at[slot], sems[slot]
        ).start()
        pltpu.make_async_copy(
            tgt_hbm.at[pl.ds(off, _CHUNK)], tbuf.at[slot], sems[slot]
        ).start()

    def wait(slot):
        pltpu.make_async_copy(
            pred_hbm.at[pl.ds(base, _CHUNK)], pbuf.at[slot], sems[slot]
        ).wait()
        pltpu.make_async_copy(
            tgt_hbm.at[pl.ds(base, _CHUNK)], tbuf.at[slot], sems[slot]
        ).wait()

    def compute(slot, accs):
        pb = pbuf.at[slot]
        tb = tbuf.at[slot]

        def vec_body(off, accs):
            new = []
            for u in range(_UNROLL):
                p = pb[pl.ds(off + u * _L, _L)]
                t = tb[pl.ds(off + u * _L, _L)]
                d = p - t
                w = wt[0]
                for k in range(4):
                    w = jnp.where(t >= th[k], wt[k + 1], w)
                new.append(accs[u] + w * (d * d))
            return tuple(new)

        return plsc.parallel_loop(
            0, _CHUNK, step=_UNROLL * _L, unroll=2, carry=accs
        )(vec_body)

    zero = jnp.zeros((_L,), jnp.float32)
    accs = (zero,) * _UNROLL

    start(0, 0)
    start(1, 1)

    def pair_body(j, accs):
        c = 2 * j
        wait(0)
        accs = compute(0, accs)
        start(c + 2, 0)
        wait(1)
        accs = compute(1, accs)
        start(c + 3, 1)
        return accs

    accs = lax.fori_loop(0, n_chunks // 2 - 1, pair_body, accs)
    wait(0)
    accs = compute(0, accs)
    wait(1)
    accs = compute(1, accs)

    tot = accs[0]
    for u in range(1, _UNROLL):
        tot = tot + accs[u]
    stage[...] = tot
    pltpu.sync_copy(stage, out_hbm.at[pl.ds(wid * _L, _L)])


def _tc_body(pred_ref, tgt_ref, w_ref, t_ref, out_ref, acc_ref):
    i = pl.program_id(0)
    n = pl.num_programs(0)

    t = tgt_ref[...]
    p = pred_ref[...]
    d = p - t
    sq = d * d
    w = jnp.full_like(t, w_ref[0])
    for k in range(4):
        w = jnp.where(t >= t_ref[k], w_ref[k + 1], w)
    partial = jnp.sum(w * sq, axis=0)

    @pl.when(i == 0)
    def _init():
        acc_ref[...] = jnp.zeros_like(acc_ref)

    acc_ref[...] += partial.reshape(acc_ref.shape)

    @pl.when(i == n - 1)
    def _fin():
        out_ref[0] = jnp.sum(acc_ref[...])


def kernel(prediction, target, weights, thresholds):
    total = prediction.size
    rows = total // 1024
    tc_rows = rows - _SC_ROWS
    p2 = prediction.reshape(rows, 1024)
    t2 = target.reshape(rows, 1024)
    p1 = prediction.reshape(total)
    t1 = target.reshape(total)

    consts = jnp.concatenate([thresholds, weights]).reshape(9, 1) * jnp.ones(
        (1, _L), jnp.float32
    )

    sc_fn = functools.partial(
        pl.kernel,
        mesh=plsc.VectorSubcoreMesh(core_axis_name="c", subcore_axis_name="s"),
        out_type=jax.ShapeDtypeStruct((_NW * _L,), jnp.float32),
        scratch_types=[
            pltpu.VMEM((2, _CHUNK), jnp.float32),
            pltpu.VMEM((2, _CHUNK), jnp.float32),
            pltpu.VMEM((9, _L), jnp.float32),
            pltpu.VMEM((_L,), jnp.float32),
            pltpu.SemaphoreType.DMA,
            pltpu.SemaphoreType.DMA,
        ],
    )(functools.partial(_sc_body, tc_rows * 1024))
    sc_partials = sc_fn(p1, t1, consts)

    tc_part = pl.pallas_call(
        _tc_body,
        grid=(tc_rows // _TC_BLOCK_ROWS,),
        in_specs=[
            pl.BlockSpec((_TC_BLOCK_ROWS, 1024), lambda i: (i, 0)),
            pl.BlockSpec((_TC_BLOCK_ROWS, 1024), lambda i: (i, 0)),
            pl.BlockSpec(memory_space=pltpu.SMEM),
            pl.BlockSpec(memory_space=pltpu.SMEM),
        ],
        out_specs=pl.BlockSpec(memory_space=pltpu.SMEM),
        out_shape=jax.ShapeDtypeStruct((1,), jnp.float32),
        scratch_shapes=[pltpu.VMEM((8, 128), jnp.float32)],
    )(p2, t2, weights, thresholds)

    s = tc_part[0] + jnp.sum(sc_partials)
    return (s / total).astype(jnp.float32).reshape(())


# speedup vs baseline: 2.7504x; 1.6460x over previous
import functools

import jax
import jax.numpy as jnp
from jax import lax
from jax.experimental import pallas as pl
from jax.experimental.pallas import tpu as pltpu
from jax.experimental.pallas import tpu_sc as plsc

_NC = 2
_NS = 16
_NW = _NC * _NS
_L = 16
_CROWS = 16
_UNROLL = 8


def _sc_body(
    pred_hbm, tgt_hbm, consts_hbm, out_hbm, pbuf, tbuf, cbuf, stage, sem0, sem1
):
    wid = lax.axis_index("s") * _NC + lax.axis_index("c")
    rows = pred_hbm.shape[0]
    rpw = rows // _NW
    n_chunks = rpw // _CROWS
    base = wid * rpw
    sems = (sem0, sem1)

    pltpu.sync_copy(consts_hbm, cbuf)
    th = [cbuf[i] for i in range(4)]
    wt = [cbuf[4 + i] for i in range(5)]

    def start(chunk, slot):
        r0 = base + chunk * _CROWS
        pltpu.make_async_copy(
            pred_hbm.at[pl.ds(r0, _CROWS)], pbuf.at[slot], sems[slot]
        ).start()
        pltpu.make_async_copy(
            tgt_hbm.at[pl.ds(r0, _CROWS)], tbuf.at[slot], sems[slot]
        ).start()

    def wait(slot):
        pltpu.make_async_copy(
            pred_hbm.at[pl.ds(base, _CROWS)], pbuf.at[slot], sems[slot]
        ).wait()
        pltpu.make_async_copy(
            tgt_hbm.at[pl.ds(base, _CROWS)], tbuf.at[slot], sems[slot]
        ).wait()

    def compute(slot, accs):
        pb = pbuf.at[slot]
        tb = tbuf.at[slot]

        def row_body(r, accs):
            def vec_body(off, accs):
                new = []
                for u in range(_UNROLL):
                    p = pb[r, pl.ds(off + u * _L, _L)]
                    t = tb[r, pl.ds(off + u * _L, _L)]
                    d = p - t
                    w = wt[0]
                    for k in range(4):
                        w = jnp.where(t >= th[k], wt[k + 1], w)
                    new.append(accs[u] + w * (d * d))
                return tuple(new)

            return plsc.parallel_loop(
                0, 1024, step=_UNROLL * _L, unroll=2, carry=accs
            )(vec_body)

        return lax.fori_loop(0, _CROWS, row_body, accs)

    zero = jnp.zeros((_L,), jnp.float32)
    accs = (zero,) * _UNROLL

    start(0, 0)
    start(1, 1)

    def pair_body(j, accs):
        c = 2 * j
        wait(0)
        accs = compute(0, accs)
        start(c + 2, 0)
        wait(1)
        accs = compute(1, accs)
        start(c + 3, 1)
        return accs

    accs = lax.fori_loop(0, n_chunks // 2 - 1, pair_body, accs)
    wait(0)
    accs = compute(0, accs)
    wait(1)
    accs = compute(1, accs)

    tot = accs[0]
    for u in range(1, _UNROLL):
        tot = tot + accs[u]
    stage[...] = tot
    pltpu.sync_copy(stage, out_hbm.at[pl.ds(wid * _L, _L)])


def kernel(prediction, target, weights, thresholds):
    total = prediction.size
    rows = total // 1024
    p2 = prediction.reshape(rows, 1024)
    t2 = target.reshape(rows, 1024)

    consts = jnp.concatenate([thresholds, weights]).reshape(9, 1) * jnp.ones(
        (1, _L), jnp.float32
    )

    sc_fn = functools.partial(
        pl.kernel,
        mesh=plsc.VectorSubcoreMesh(core_axis_name="c", subcore_axis_name="s"),
        out_type=jax.ShapeDtypeStruct((_NW * _L,), jnp.float32),
        scratch_types=[
            pltpu.VMEM((2, _CROWS, 1024), jnp.float32),
            pltpu.VMEM((2, _CROWS, 1024), jnp.float32),
            pltpu.VMEM((9, _L), jnp.float32),
            pltpu.VMEM((_L,), jnp.float32),
            pltpu.SemaphoreType.DMA,
            pltpu.SemaphoreType.DMA,
        ],
    )(_sc_body)
    partials = sc_fn(p2, t2, consts)
    return (jnp.sum(partials) / total).astype(jnp.float32).reshape(())


# speedup vs baseline: 4.6705x; 1.6981x over previous
import functools

import jax
import jax.numpy as jnp
from jax import lax
from jax.experimental import pallas as pl
from jax.experimental.pallas import tpu as pltpu
from jax.experimental.pallas import tpu_sc as plsc

_NC = 2
_NS = 16
_NW = _NC * _NS
_L = 16
_CROWS = 16
_UNROLL = 8

_SC_ROWS = 12288
_TC_BLOCK_ROWS = 512


def _sc_body(
    sc_base_row, pred_hbm, tgt_hbm, consts_hbm, out_hbm,
    pbuf, tbuf, cbuf, stage, sem0, sem1,
):
    wid = lax.axis_index("s") * _NC + lax.axis_index("c")
    rpw = _SC_ROWS // _NW
    n_chunks = rpw // _CROWS
    base = sc_base_row + wid * rpw
    sems = (sem0, sem1)

    pltpu.sync_copy(consts_hbm, cbuf)
    th = [cbuf[i] for i in range(4)]
    wt = [cbuf[4 + i] for i in range(5)]

    def start(chunk, slot):
        r0 = base + chunk * _CROWS
        pltpu.make_async_copy(
            pred_hbm.at[pl.ds(r0, _CROWS)], pbuf.at[slot], sems[slot]
        ).start()
        pltpu.make_async_copy(
            tgt_hbm.at[pl.ds(r0, _CROWS)], tbuf.at[slot], sems[slot]
        ).start()

    def wait(slot):
        pltpu.make_async_copy(
            pred_hbm.at[pl.ds(base, _CROWS)], pbuf.at[slot], sems[slot]
        ).wait()
        pltpu.make_async_copy(
            tgt_hbm.at[pl.ds(base, _CROWS)], tbuf.at[slot], sems[slot]
        ).wait()

    def compute(slot, accs):
        pb = pbuf.at[slot]
        tb = tbuf.at[slot]

        def row_body(r, accs):
            def vec_body(off, accs):
                new = []
                for u in range(_UNROLL):
                    p = pb[r, pl.ds(off + u * _L, _L)]
                    t = tb[r, pl.ds(off + u * _L, _L)]
                    d = p - t
                    w = wt[0]
                    for k in range(4):
                        w = jnp.where(t >= th[k], wt[k + 1], w)
                    new.append(accs[u] + w * (d * d))
                return tuple(new)

            return plsc.parallel_loop(
                0, 1024, step=_UNROLL * _L, unroll=2, carry=accs
            )(vec_body)

        return lax.fori_loop(0, _CROWS, row_body, accs)

    zero = jnp.zeros((_L,), jnp.float32)
    accs = (zero,) * _UNROLL

    start(0, 0)
    start(1, 1)

    def pair_body(j, accs):
        c = 2 * j
        wait(0)
        accs = compute(0, accs)
        start(c + 2, 0)
        wait(1)
        accs = compute(1, accs)
        start(c + 3, 1)
        return accs

    accs = lax.fori_loop(0, n_chunks // 2 - 1, pair_body, accs)
    wait(0)
    accs = compute(0, accs)
    wait(1)
    accs = compute(1, accs)

    tot = accs[0]
    for u in range(1, _UNROLL):
        tot = tot + accs[u]
    stage[...] = tot
    pltpu.sync_copy(stage, out_hbm.at[pl.ds(wid * _L, _L)])


def _tc_body(pred_ref, tgt_ref, w_ref, t_ref, out_ref, acc_ref):
    i = pl.program_id(0)
    n = pl.num_programs(0)

    t = tgt_ref[...]
    p = pred_ref[...]
    d = p - t
    sq = d * d
    w = jnp.full_like(t, w_ref[0])
    for k in range(4):
        w = jnp.where(t >= t_ref[k], w_ref[k + 1], w)
    partial = jnp.sum(w * sq, axis=0)

    @pl.when(i == 0)
    def _init():
        acc_ref[...] = jnp.zeros_like(acc_ref)

    acc_ref[...] += partial.reshape(acc_ref.shape)

    @pl.when(i == n - 1)
    def _fin():
        out_ref[0] = jnp.sum(acc_ref[...])


def kernel(prediction, target, weights, thresholds):
    total = prediction.size
    rows = total // 1024
    tc_rows = rows - _SC_ROWS
    p2 = prediction.reshape(rows, 1024)
    t2 = target.reshape(rows, 1024)

    consts = jnp.concatenate([thresholds, weights]).reshape(9, 1) * jnp.ones(
        (1, _L), jnp.float32
    )

    sc_fn = functools.partial(
        pl.kernel,
        mesh=plsc.VectorSubcoreMesh(core_axis_name="c", subcore_axis_name="s"),
        out_type=jax.ShapeDtypeStruct((_NW * _L,), jnp.float32),
        scratch_types=[
            pltpu.VMEM((2, _CROWS, 1024), jnp.float32),
            pltpu.VMEM((2, _CROWS, 1024), jnp.float32),
            pltpu.VMEM((9, _L), jnp.float32),
            pltpu.VMEM((_L,), jnp.float32),
            pltpu.SemaphoreType.DMA,
            pltpu.SemaphoreType.DMA,
        ],
    )(functools.partial(_sc_body, tc_rows))
    sc_partials = sc_fn(p2, t2, consts)

    tc_part = pl.pallas_call(
        _tc_body,
        grid=(tc_rows // _TC_BLOCK_ROWS,),
        in_specs=[
            pl.BlockSpec((_TC_BLOCK_ROWS, 1024), lambda i: (i, 0)),
            pl.BlockSpec((_TC_BLOCK_ROWS, 1024), lambda i: (i, 0)),
            pl.BlockSpec(memory_space=pltpu.SMEM),
            pl.BlockSpec(memory_space=pltpu.SMEM),
        ],
        out_specs=pl.BlockSpec(memory_space=pltpu.SMEM),
        out_shape=jax.ShapeDtypeStruct((1,), jnp.float32),
        scratch_shapes=[pltpu.VMEM((8, 128), jnp.float32)],
    )(p2, t2, weights, thresholds)

    s = tc_part[0] + jnp.sum(sc_partials)
    return (s / total).astype(jnp.float32).reshape(())


# speedup vs baseline: 4.7560x; 1.0183x over previous
import functools

import jax
import jax.numpy as jnp
from jax import lax
from jax.experimental import pallas as pl
from jax.experimental.pallas import tpu as pltpu
from jax.experimental.pallas import tpu_sc as plsc

_NC = 2
_NS = 16
_NW = _NC * _NS
_L = 16
_CROWS = 16
_UNROLL = 8

_SC_ROWS = 13312
_TC_BLOCK_ROWS = 512


def _sc_body(
    sc_base_row, pred_hbm, tgt_hbm, consts_hbm, out_hbm,
    pbuf, tbuf, cbuf, stage, sem0, sem1,
):
    wid = lax.axis_index("s") * _NC + lax.axis_index("c")
    rpw = _SC_ROWS // _NW
    n_chunks = rpw // _CROWS
    base = sc_base_row + wid * rpw
    sems = (sem0, sem1)

    pltpu.sync_copy(consts_hbm, cbuf)
    th = [cbuf[i] for i in range(4)]
    wt = [cbuf[4 + i] for i in range(5)]

    def start(chunk, slot):
        r0 = base + chunk * _CROWS
        pltpu.make_async_copy(
            pred_hbm.at[pl.ds(r0, _CROWS)], pbuf.at[slot], sems[slot]
        ).start()
        pltpu.make_async_copy(
            tgt_hbm.at[pl.ds(r0, _CROWS)], tbuf.at[slot], sems[slot]
        ).start()

    def wait(slot):
        pltpu.make_async_copy(
            pred_hbm.at[pl.ds(base, _CROWS)], pbuf.at[slot], sems[slot]
        ).wait()
        pltpu.make_async_copy(
            tgt_hbm.at[pl.ds(base, _CROWS)], tbuf.at[slot], sems[slot]
        ).wait()

    def compute(slot, accs):
        pb = pbuf.at[slot]
        tb = tbuf.at[slot]

        def row_body(r, accs):
            def vec_body(off, accs):
                new = []
                for u in range(_UNROLL):
                    p = pb[r, pl.ds(off + u * _L, _L)]
                    t = tb[r, pl.ds(off + u * _L, _L)]
                    d = p - t
                    w = wt[0]
                    for k in range(4):
                        w = jnp.where(t >= th[k], wt[k + 1], w)
                    new.append(accs[u] + w * (d * d))
                return tuple(new)

            return plsc.parallel_loop(
                0, 1024, step=_UNROLL * _L, unroll=2, carry=accs
            )(vec_body)

        return lax.fori_loop(0, _CROWS, row_body, accs)

    zero = jnp.zeros((_L,), jnp.float32)
    accs = (zero,) * _UNROLL

    start(0, 0)
    start(1, 1)

    def pair_body(j, accs):
        c = 2 * j
        wait(0)
        accs = compute(0, accs)
        start(c + 2, 0)
        wait(1)
        accs = compute(1, accs)
        start(c + 3, 1)
        return accs

    accs = lax.fori_loop(0, n_chunks // 2 - 1, pair_body, accs)
    wait(0)
    accs = compute(0, accs)
    wait(1)
    accs = compute(1, accs)

    tot = accs[0]
    for u in range(1, _UNROLL):
        tot = tot + accs[u]
    stage[...] = tot
    pltpu.sync_copy(stage, out_hbm.at[pl.ds(wid * _L, _L)])


def _tc_body(pred_ref, tgt_ref, w_ref, t_ref, out_ref, acc_ref):
    i = pl.program_id(0)
    n = pl.num_programs(0)

    t = tgt_ref[...]
    p = pred_ref[...]
    d = p - t
    sq = d * d
    w = jnp.full_like(t, w_ref[0])
    for k in range(4):
        w = jnp.where(t >= t_ref[k], w_ref[k + 1], w)
    partial = jnp.sum(w * sq, axis=0)

    @pl.when(i == 0)
    def _init():
        acc_ref[...] = jnp.zeros_like(acc_ref)

    acc_ref[...] += partial.reshape(acc_ref.shape)

    @pl.when(i == n - 1)
    def _fin():
        out_ref[0] = jnp.sum(acc_ref[...])


def kernel(prediction, target, weights, thresholds):
    total = prediction.size
    rows = total // 1024
    tc_rows = rows - _SC_ROWS
    p2 = prediction.reshape(rows, 1024)
    t2 = target.reshape(rows, 1024)

    consts = jnp.concatenate([thresholds, weights]).reshape(9, 1) * jnp.ones(
        (1, _L), jnp.float32
    )

    sc_fn = functools.partial(
        pl.kernel,
        mesh=plsc.VectorSubcoreMesh(core_axis_name="c", subcore_axis_name="s"),
        out_type=jax.ShapeDtypeStruct((_NW * _L,), jnp.float32),
        scratch_types=[
            pltpu.VMEM((2, _CROWS, 1024), jnp.float32),
            pltpu.VMEM((2, _CROWS, 1024), jnp.float32),
            pltpu.VMEM((9, _L), jnp.float32),
            pltpu.VMEM((_L,), jnp.float32),
            pltpu.SemaphoreType.DMA,
            pltpu.SemaphoreType.DMA,
        ],
    )(functools.partial(_sc_body, tc_rows))
    sc_partials = sc_fn(p2, t2, consts)

    tc_part = pl.pallas_call(
        _tc_body,
        grid=(tc_rows // _TC_BLOCK_ROWS,),
        in_specs=[
            pl.BlockSpec((_TC_BLOCK_ROWS, 1024), lambda i: (i, 0)),
            pl.BlockSpec((_TC_BLOCK_ROWS, 1024), lambda i: (i, 0)),
            pl.BlockSpec(memory_space=pltpu.SMEM),
            pl.BlockSpec(memory_space=pltpu.SMEM),
        ],
        out_specs=pl.BlockSpec(memory_space=pltpu.SMEM),
        out_shape=jax.ShapeDtypeStruct((1,), jnp.float32),
        scratch_shapes=[pltpu.VMEM((8, 128), jnp.float32)],
    )(p2, t2, weights, thresholds)

    s = tc_part[0] + jnp.sum(sc_partials)
    return (s / total).astype(jnp.float32).reshape(())


# speedup vs baseline: 4.8359x; 1.0168x over previous
import functools

import jax
import jax.numpy as jnp
from jax import lax
from jax.experimental import pallas as pl
from jax.experimental.pallas import tpu as pltpu
from jax.experimental.pallas import tpu_sc as plsc

_NC = 2
_NS = 16
_NW = _NC * _NS
_L = 16
_CROWS = 16
_UNROLL = 8

_SC_ROWS = 14336
_TC_BLOCK_ROWS = 512


def _sc_body(
    sc_base_row, pred_hbm, tgt_hbm, consts_hbm, out_hbm,
    pbuf, tbuf, cbuf, stage, sem0, sem1,
):
    wid = lax.axis_index("s") * _NC + lax.axis_index("c")
    rpw = _SC_ROWS // _NW
    n_chunks = rpw // _CROWS
    base = sc_base_row + wid * rpw
    sems = (sem0, sem1)

    pltpu.sync_copy(consts_hbm, cbuf)
    th = [cbuf[i] for i in range(4)]
    wt = [cbuf[4 + i] for i in range(5)]

    def start(chunk, slot):
        r0 = base + chunk * _CROWS
        pltpu.make_async_copy(
            pred_hbm.at[pl.ds(r0, _CROWS)], pbuf.at[slot], sems[slot]
        ).start()
        pltpu.make_async_copy(
            tgt_hbm.at[pl.ds(r0, _CROWS)], tbuf.at[slot], sems[slot]
        ).start()

    def wait(slot):
        pltpu.make_async_copy(
            pred_hbm.at[pl.ds(base, _CROWS)], pbuf.at[slot], sems[slot]
        ).wait()
        pltpu.make_async_copy(
            tgt_hbm.at[pl.ds(base, _CROWS)], tbuf.at[slot], sems[slot]
        ).wait()

    def compute(slot, accs):
        pb = pbuf.at[slot]
        tb = tbuf.at[slot]

        def row_body(r, accs):
            def vec_body(off, accs):
                new = []
                for u in range(_UNROLL):
                    p = pb[r, pl.ds(off + u * _L, _L)]
                    t = tb[r, pl.ds(off + u * _L, _L)]
                    d = p - t
                    w = wt[0]
                    for k in range(4):
                        w = jnp.where(t >= th[k], wt[k + 1], w)
                    new.append(accs[u] + w * (d * d))
                return tuple(new)

            return plsc.parallel_loop(
                0, 1024, step=_UNROLL * _L, unroll=2, carry=accs
            )(vec_body)

        return lax.fori_loop(0, _CROWS, row_body, accs)

    zero = jnp.zeros((_L,), jnp.float32)
    accs = (zero,) * _UNROLL

    start(0, 0)
    start(1, 1)

    def pair_body(j, accs):
        c = 2 * j
        wait(0)
        accs = compute(0, accs)
        start(c + 2, 0)
        wait(1)
        accs = compute(1, accs)
        start(c + 3, 1)
        return accs

    accs = lax.fori_loop(0, n_chunks // 2 - 1, pair_body, accs)
    wait(0)
    accs = compute(0, accs)
    wait(1)
    accs = compute(1, accs)

    tot = accs[0]
    for u in range(1, _UNROLL):
        tot = tot + accs[u]
    stage[...] = tot
    pltpu.sync_copy(stage, out_hbm.at[pl.ds(wid * _L, _L)])


def _tc_body(pred_ref, tgt_ref, w_ref, t_ref, out_ref, acc_ref):
    i = pl.program_id(0)
    n = pl.num_programs(0)

    t = tgt_ref[...]
    p = pred_ref[...]
    d = p - t
    sq = d * d
    w = jnp.full_like(t, w_ref[0])
    for k in range(4):
        w = jnp.where(t >= t_ref[k], w_ref[k + 1], w)
    partial = jnp.sum(w * sq, axis=0)

    @pl.when(i == 0)
    def _init():
        acc_ref[...] = jnp.zeros_like(acc_ref)

    acc_ref[...] += partial.reshape(acc_ref.shape)

    @pl.when(i == n - 1)
    def _fin():
        out_ref[0] = jnp.sum(acc_ref[...])


def kernel(prediction, target, weights, thresholds):
    total = prediction.size
    rows = total // 1024
    tc_rows = rows - _SC_ROWS
    p2 = prediction.reshape(rows, 1024)
    t2 = target.reshape(rows, 1024)

    consts = jnp.concatenate([thresholds, weights]).reshape(9, 1) * jnp.ones(
        (1, _L), jnp.float32
    )

    sc_fn = functools.partial(
        pl.kernel,
        mesh=plsc.VectorSubcoreMesh(core_axis_name="c", subcore_axis_name="s"),
        out_type=jax.ShapeDtypeStruct((_NW * _L,), jnp.float32),
        scratch_types=[
            pltpu.VMEM((2, _CROWS, 1024), jnp.float32),
            pltpu.VMEM((2, _CROWS, 1024), jnp.float32),
            pltpu.VMEM((9, _L), jnp.float32),
            pltpu.VMEM((_L,), jnp.float32),
            pltpu.SemaphoreType.DMA,
            pltpu.SemaphoreType.DMA,
        ],
    )(functools.partial(_sc_body, tc_rows))
    sc_partials = sc_fn(p2, t2, consts)

    tc_part = pl.pallas_call(
        _tc_body,
        grid=(tc_rows // _TC_BLOCK_ROWS,),
        in_specs=[
            pl.BlockSpec((_TC_BLOCK_ROWS, 1024), lambda i: (i, 0)),
            pl.BlockSpec((_TC_BLOCK_ROWS, 1024), lambda i: (i, 0)),
            pl.BlockSpec(memory_space=pltpu.SMEM),
            pl.BlockSpec(memory_space=pltpu.SMEM),
        ],
        out_specs=pl.BlockSpec(memory_space=pltpu.SMEM),
        out_shape=jax.ShapeDtypeStruct((1,), jnp.float32),
        scratch_shapes=[pltpu.VMEM((8, 128), jnp.float32)],
    )(p2, t2, weights, thresholds)

    s = tc_part[0] + jnp.sum(sc_partials)
    return (s / total).astype(jnp.float32).reshape(())
